# Initial kernel scaffold; baseline (speedup 1.0000x reference)
import jax
import jax.numpy as jnp
from jax.experimental import pallas as pl


def kernel(feat, bit_position, edge_index, W_self1, b_self1, W_self2, b_self2, Wn, bn, a_m, Wo1, bo1, Wo2, bo2):
    raise NotImplementedError("write your pallas kernel here")


# speedup vs baseline: 6.3761x; 6.3761x over previous
import jax
import jax.numpy as jnp
from jax import lax
from jax.experimental import pallas as pl
from jax.experimental.pallas import tpu as pltpu
from jax.experimental.pallas import tpu_sc as plsc

N = 10000
NP = 10240
E = 320000
FEAT = 128
H = 256
HH = 128
NC = 2
NS = 16
CH = 80
ROWB = NP // NS
BLK = 1280
AROW = NP // 128
BROW = NP // 64


def _lrelu(x, s):
    return jnp.maximum(x, s * x)


def _pre_body(x_ref, w1_ref, b1_ref, w2_ref, b2_ref, wh_ref, wf_ref, bn_ref,
              cbit_ref, am_ref,
              a0_ref, a1_ref, bp_ref, aa_ref, abp_ref, consts_ref):
    i = pl.program_id(0)
    x = x_ref[...]
    t = jnp.dot(x, w1_ref[...], preferred_element_type=jnp.float32) + b1_ref[...]
    t = _lrelu(t, 0.1)
    h = jnp.dot(t, w2_ref[...], preferred_element_type=jnp.float32) + b2_ref[...]
    a = jnp.dot(h, wh_ref[...], preferred_element_type=jnp.float32)
    bp = jnp.dot(x, wf_ref[...], preferred_element_type=jnp.float32) + bn_ref[...]
    a0_ref[...] = a[:, :HH]
    a1_ref[...] = a[:, HH:]
    bp_ref[...] = bp
    am = am_ref[...]
    aa = jnp.dot(a, am, preferred_element_type=jnp.float32)
    abp = jnp.dot(bp, am, preferred_element_type=jnp.float32)
    aa_ref[...] = aa
    abp_ref[...] = abp
    c1 = jnp.sum(cbit_ref[...] * am[:, 0][None, :])
    prevm = jnp.where(i == 0, -jnp.inf, consts_ref[0, 0])
    m = jnp.maximum(prevm, jnp.max(aa))
    lane = lax.broadcasted_iota(jnp.int32, (1, 128), 1)
    consts_ref[...] = jnp.where(lane == 0, m, jnp.where(lane == 1, c1, 0.0))


def _run_pre(featp, W1, b1, W2, b2, Wh, Wf, bnr, cbit, a_m):
    nb = NP // BLK
    full = lambda s: pl.BlockSpec(s, lambda i: (0, 0))
    return pl.pallas_call(
        _pre_body,
        grid=(nb,),
        in_specs=[
            pl.BlockSpec((BLK, FEAT), lambda i: (i, 0)),
            full((FEAT, HH)), full((1, HH)), full((HH, H)), full((1, H)),
            full((H, H)), full((FEAT, H)), full((1, H)), full((1, H)),
            full((H, 1)),
        ],
        out_specs=[
            pl.BlockSpec((BLK, HH), lambda i: (i, 0)),
            pl.BlockSpec((BLK, HH), lambda i: (i, 0)),
            pl.BlockSpec((BLK, H), lambda i: (i, 0)),
            pl.BlockSpec((BLK, 1), lambda i: (i, 0)),
            pl.BlockSpec((BLK, 1), lambda i: (i, 0)),
            pl.BlockSpec((1, 128), lambda i: (0, 0)),
        ],
        out_shape=[
            jax.ShapeDtypeStruct((NP, HH), jnp.float32),
            jax.ShapeDtypeStruct((NP, HH), jnp.float32),
            jax.ShapeDtypeStruct((NP, H), jnp.float32),
            jax.ShapeDtypeStruct((NP, 1), jnp.float32),
            jax.ShapeDtypeStruct((NP, 1), jnp.float32),
            jax.ShapeDtypeStruct((1, 128), jnp.float32),
        ],
    )(featp, W1, b1, W2, b2, Wh, Wf, bnr, cbit, a_m)


def _passa_body(src_hbm, dst_hbm, bit_hbm, aa_hbm, abp_hbm, consts_hbm,
                out_hbm, q_hbm,
                aa_v, abp_v, consts_v, srcb, dstb, bitb, rowb, qb, qrows,
                table):
    c = lax.axis_index("c")
    s = lax.axis_index("s")
    w = s * NC + c
    epert = E // (NC * NS)
    base_e = w * epert

    pltpu.sync_copy(aa_hbm, aa_v)
    pltpu.sync_copy(abp_hbm, abp_v)
    pltpu.sync_copy(consts_hbm, consts_v)

    @pl.loop(0, CH)
    def _zero(i):
        for t in range(128 // 16):
            qrows[i, pl.ds(t * 16, 16)] = jnp.zeros((16,), jnp.float32)

    @pl.when(s < 5)
    def _ztab():
        pltpu.sync_copy(qrows.at[pl.ds(0, 16)], table.at[pl.ds(s * 16, 16)])

    plsc.subcore_barrier()

    cv = consts_v[pl.ds(0, 16)]
    g = cv[0] + jnp.maximum(cv[1], 0.0)
    c1 = cv[1]

    @pl.loop(0, epert // CH)
    def _chunk(k):
        base = base_e + k * CH
        pltpu.sync_copy(src_hbm.at[pl.ds(base, CH)], srcb)
        pltpu.sync_copy(dst_hbm.at[pl.ds(base, CH)], dstb)
        pltpu.sync_copy(bit_hbm.at[pl.ds(base, CH)], bitb)
        for grp in range(CH // 16):
            rid = lax.iota(jnp.int32, 16) + grp * 16
            idxs = srcb[pl.ds(grp * 16, 16)]
            idxd = dstb[pl.ds(grp * 16, 16)]
            bitv = bitb[pl.ds(grp * 16, 16)]
            av = plsc.load_gather(aa_v, [idxs])
            bv = plsc.load_gather(abp_v, [idxd])
            u = av + bv + c1 * bitv
            e = jnp.maximum(u, 0.2 * u)
            capv = _lrelu(g + bv, 0.2)
            q = jnp.exp((e - capv) * 0.25)
            qb[pl.ds(grp * 16, 16)] = q
            rowb[pl.ds(grp * 16, 16)] = lax.shift_right_logical(idxd, 7)
            plsc.store_scatter(qrows, [rid, jnp.bitwise_and(idxd, 127)], q)
        pltpu.sync_copy(qrows, table.at[rowb], add=True)
        pltpu.sync_copy(qb, q_hbm.at[pl.ds(base, CH)])
        for grp in range(CH // 16):
            rid = lax.iota(jnp.int32, 16) + grp * 16
            idxd = dstb[pl.ds(grp * 16, 16)]
            plsc.store_scatter(qrows, [rid, jnp.bitwise_and(idxd, 127)],
                               jnp.zeros((16,), jnp.float32))

    plsc.subcore_barrier()

    @pl.when(s < 5)
    def _out():
        pltpu.sync_copy(table.at[pl.ds(s * 16, 16)],
                        out_hbm.at[c, pl.ds(s * 16, 16)])


def _run_passa(src, dst, bit, aa, abp, consts):
    mesh = plsc.VectorSubcoreMesh(core_axis_name="c", subcore_axis_name="s",
                                  num_cores=NC, num_subcores=NS)
    f = pl.kernel(
        _passa_body,
        out_type=(jax.ShapeDtypeStruct((NC, AROW, 128), jnp.float32),
                  jax.ShapeDtypeStruct((E,), jnp.float32)),
        mesh=mesh,
        compiler_params=pltpu.CompilerParams(needs_layout_passes=False),
        scratch_types=[
            pltpu.VMEM((NP,), jnp.float32),
            pltpu.VMEM((NP,), jnp.float32),
            pltpu.VMEM((128,), jnp.float32),
            pltpu.VMEM((CH,), jnp.int32),
            pltpu.VMEM((CH,), jnp.int32),
            pltpu.VMEM((CH,), jnp.float32),
            pltpu.VMEM((CH,), jnp.int32),
            pltpu.VMEM((CH,), jnp.float32),
            pltpu.VMEM((CH, 128), jnp.float32),
            pltpu.VMEM_SHARED((AROW, 128), jnp.float32),
        ],
    )
    return f(src, dst, bit, aa, abp, consts)


def _mid_body(s0a0_ref, s0a1_ref, sinv_ref):
    stot = s0a0_ref[0] + s0a1_ref[0]
    pos = stot > 0
    sinv_ref[...] = jnp.where(pos, 1.0 / jnp.where(pos, stot, 1.0), 0.0)


def _run_mid(s0a):
    return pl.pallas_call(
        _mid_body,
        grid=(1,),
        in_specs=[
            pl.BlockSpec((1, AROW, 128), lambda i: (0, 0, 0)),
            pl.BlockSpec((1, AROW, 128), lambda i: (1, 0, 0)),
        ],
        out_specs=pl.BlockSpec((AROW, 128), lambda i: (0, 0)),
        out_shape=jax.ShapeDtypeStruct((AROW, 128), jnp.float32),
    )(s0a, s0a)


def _passb_body(src_hbm, dst_hbm, bit_hbm, q_hbm, a0_hbm, a1_hbm, sinv_hbm,
                out_hbm, outw_hbm,
                sinv_v, srcb, dstb, bitb, qb, rowb, rows, wrows, table,
                wtable, sem):
    c = lax.axis_index("c")
    s = lax.axis_index("s")
    epert = E // NS
    base_e = s * epert

    pltpu.sync_copy(sinv_hbm, sinv_v)

    @pl.loop(0, CH)
    def _zero(i):
        for t in range(HH // 16):
            rows[i, pl.ds(t * 16, 16)] = jnp.zeros((16,), jnp.float32)
        for t in range(128 // 16):
            wrows[i, pl.ds(t * 16, 16)] = jnp.zeros((16,), jnp.float32)

    for j in range(ROWB // CH):
        pltpu.sync_copy(rows, table.at[pl.ds(s * ROWB + j * CH, CH)])

    @pl.when(s < BROW // 16)
    def _ztab():
        pltpu.sync_copy(wrows.at[pl.ds(0, 16)], wtable.at[pl.ds(s * 16, 16)])

    plsc.subcore_barrier()

    @pl.loop(0, epert // CH)
    def _chunk(k):
        base = base_e + k * CH
        pltpu.sync_copy(src_hbm.at[pl.ds(base, CH)], srcb)
        pltpu.sync_copy(dst_hbm.at[pl.ds(base, CH)], dstb)
        pltpu.sync_copy(bit_hbm.at[pl.ds(base, CH)], bitb)
        pltpu.sync_copy(q_hbm.at[pl.ds(base, CH)], qb)

        @pl.when(c == 0)
        def _g0():
            pltpu.async_copy(a0_hbm.at[srcb], rows, sem).wait()

        @pl.when(c != 0)
        def _g1():
            pltpu.async_copy(a1_hbm.at[srcb], rows, sem).wait()

        for grp in range(CH // 16):
            rid = lax.iota(jnp.int32, 16) + grp * 16
            idxd = dstb[pl.ds(grp * 16, 16)]
            bitv = bitb[pl.ds(grp * 16, 16)]
            qv = qb[pl.ds(grp * 16, 16)]
            sv = plsc.load_gather(sinv_v, [idxd])
            t = qv * sv
            t2 = t * t
            wv = t2 * t2
            rowb[pl.ds(grp * 16, 16)] = lax.shift_right_logical(idxd, 6)
            wcol = jnp.bitwise_and(idxd, 63) * 2
            plsc.store_scatter(wrows, [rid, wcol], wv)
            plsc.store_scatter(wrows, [rid, wcol + 1], wv * bitv)
            for lane in range(16):
                ei = grp * 16 + lane
                wsc = wv[lane]
                for t in range(HH // 16):
                    rows[ei, pl.ds(t * 16, 16)] = (
                        rows[ei, pl.ds(t * 16, 16)] * wsc)
        pltpu.sync_copy(rows, table.at[dstb], add=True)
        pltpu.sync_copy(wrows, wtable.at[rowb], add=True)
        for grp in range(CH // 16):
            rid = lax.iota(jnp.int32, 16) + grp * 16
            idxd = dstb[pl.ds(grp * 16, 16)]
            wcol = jnp.bitwise_and(idxd, 63) * 2
            z16 = jnp.zeros((16,), jnp.float32)
            plsc.store_scatter(wrows, [rid, wcol], z16)
            plsc.store_scatter(wrows, [rid, wcol + 1], z16)

    plsc.subcore_barrier()
    pltpu.sync_copy(table.at[pl.ds(s * ROWB, ROWB)],
                    out_hbm.at[c, pl.ds(s * ROWB, ROWB)])

    @pl.when(s < BROW // 16)
    def _outw():
        pltpu.sync_copy(wtable.at[pl.ds(s * 16, 16)],
                        outw_hbm.at[c, pl.ds(s * 16, 16)])


def _run_passb(src, dst, bit, q, a0, a1, sinv):
    mesh = plsc.VectorSubcoreMesh(core_axis_name="c", subcore_axis_name="s",
                                  num_cores=NC, num_subcores=NS)
    f = pl.kernel(
        _passb_body,
        out_type=(jax.ShapeDtypeStruct((NC, NP, HH), jnp.float32),
                  jax.ShapeDtypeStruct((NC, BROW, 128), jnp.float32)),
        mesh=mesh,
        compiler_params=pltpu.CompilerParams(needs_layout_passes=False),
        scratch_types=[
            pltpu.VMEM((NP,), jnp.float32),
            pltpu.VMEM((CH,), jnp.int32),
            pltpu.VMEM((CH,), jnp.int32),
            pltpu.VMEM((CH,), jnp.float32),
            pltpu.VMEM((CH,), jnp.float32),
            pltpu.VMEM((CH,), jnp.int32),
            pltpu.VMEM((CH, HH), jnp.float32),
            pltpu.VMEM((CH, 128), jnp.float32),
            pltpu.VMEM_SHARED((NP, HH), jnp.float32),
            pltpu.VMEM_SHARED((BROW, 128), jnp.float32),
            pltpu.SemaphoreType.DMA,
        ],
    )
    return f(src, dst, bit, q, a0, a1, sinv)


def _post_body(sa0_ref, sa1_ref, s0_ref, s1_ref, bp_ref, cbit_ref, wo1_ref,
               bo1_ref, wo2_ref, bo2_ref, out_ref):
    s0 = s0_ref[...]
    s1 = s1_ref[...]
    sa = jnp.concatenate([sa0_ref[0], sa1_ref[0]], axis=1)
    inv = 1.0 / jnp.where(s0 > 0, s0, 1.0)
    neigh = jnp.where(s0 > 0, (sa + s1 * cbit_ref[...]) * inv + bp_ref[...], 0.0)
    ho = jnp.maximum(neigh, 0.0)
    t = jnp.dot(ho, wo1_ref[...], preferred_element_type=jnp.float32) + bo1_ref[...]
    t = _lrelu(t, 0.1)
    out_ref[...] = jnp.dot(t, wo2_ref[...], preferred_element_type=jnp.float32) + bo2_ref[...]


def _run_post(sab, s0, s1, bp, cbit, Wo1, bo1, Wo2, bo2):
    nb = NP // BLK
    full = lambda s: pl.BlockSpec(s, lambda i: (0, 0))
    return pl.pallas_call(
        _post_body,
        grid=(nb,),
        in_specs=[
            pl.BlockSpec((1, BLK, HH), lambda i: (0, i, 0)),
            pl.BlockSpec((1, BLK, HH), lambda i: (1, i, 0)),
            pl.BlockSpec((BLK, 1), lambda i: (i, 0)),
            pl.BlockSpec((BLK, 1), lambda i: (i, 0)),
            pl.BlockSpec((BLK, H), lambda i: (i, 0)),
            full((1, H)), full((H, H)), full((1, H)), full((H, 1)),
            full((1, 1)),
        ],
        out_specs=pl.BlockSpec((BLK, 1), lambda i: (i, 0)),
        out_shape=jax.ShapeDtypeStruct((NP, 1), jnp.float32),
    )(sab, sab, s0, s1, bp, cbit, Wo1, bo1, Wo2, bo2)


@jax.jit
def kernel(feat, bit_position, edge_index, W_self1, b_self1, W_self2, b_self2,
           Wn, bn, a_m, Wo1, bo1, Wo2, bo2):
    src = edge_index[0]
    dst = edge_index[1]
    bit = bit_position[:, 0]
    featp = jnp.pad(feat, ((0, NP - N), (0, 0)))
    Wh = Wn[:H]
    cbit = Wn[H:H + 1]
    Wf = Wn[H + 1:]
    a0, a1, bp, aa2, abp2, consts = _run_pre(
        featp, W_self1, b_self1[None, :], W_self2, b_self2[None, :],
        Wh, Wf, bn[None, :], cbit, a_m)
    aa = aa2[:, 0]
    abp = abp2[:, 0]
    constsv = consts[0]
    s0a, q = _run_passa(src, dst, bit, aa, abp, constsv)
    sinv = _run_mid(s0a).reshape(NP)
    sab, sw = _run_passb(src, dst, bit, q, a0, a1, sinv)
    sw2 = sw[0].reshape(NP, 2)
    out = _run_post(sab, sw2[:, 0:1], sw2[:, 1:2], bp, cbit,
                    Wo1, bo1[None, :], Wo2, bo2[None, :])
    return out[:N]


# speedup vs baseline: 7.2349x; 1.1347x over previous
import jax
import jax.numpy as jnp
from jax import lax
from jax.experimental import pallas as pl
from jax.experimental.pallas import tpu as pltpu
from jax.experimental.pallas import tpu_sc as plsc

N = 10000
NP = 10240
E = 320000
FEAT = 128
H = 256
HH = 128
NC = 2
NS = 16
CH = 80
ROWB = NP // NS
BLK = 1280
AROW = NP // 128
BROW = NP // 64


def _lrelu(x, s):
    return jnp.maximum(x, s * x)


def _pre_body(x_ref, w1_ref, b1_ref, w2_ref, b2_ref, wh_ref, wf_ref, bn_ref,
              cbit_ref, am_ref,
              a0_ref, a1_ref, bp_ref, aa_ref, abp_ref, consts_ref):
    i = pl.program_id(0)
    x = x_ref[...]
    t = jnp.dot(x, w1_ref[...], preferred_element_type=jnp.float32) + b1_ref[...]
    t = _lrelu(t, 0.1)
    h = jnp.dot(t, w2_ref[...], preferred_element_type=jnp.float32) + b2_ref[...]
    a = jnp.dot(h, wh_ref[...], preferred_element_type=jnp.float32)
    bp = jnp.dot(x, wf_ref[...], preferred_element_type=jnp.float32) + bn_ref[...]
    a0_ref[...] = a[:, :HH]
    a1_ref[...] = a[:, HH:]
    bp_ref[...] = bp
    am = am_ref[...]
    aa = jnp.dot(a, am, preferred_element_type=jnp.float32)
    abp = jnp.dot(bp, am, preferred_element_type=jnp.float32)
    aa_ref[...] = aa
    abp_ref[...] = abp
    c1 = jnp.sum(cbit_ref[...] * am[:, 0][None, :])
    prevm = jnp.where(i == 0, -jnp.inf, consts_ref[0, 0])
    m = jnp.maximum(prevm, jnp.max(aa))
    lane = lax.broadcasted_iota(jnp.int32, (1, 128), 1)
    consts_ref[...] = jnp.where(lane == 0, m, jnp.where(lane == 1, c1, 0.0))


def _run_pre(featp, W1, b1, W2, b2, Wh, Wf, bnr, cbit, a_m):
    nb = NP // BLK
    full = lambda s: pl.BlockSpec(s, lambda i: (0, 0))
    return pl.pallas_call(
        _pre_body,
        grid=(nb,),
        in_specs=[
            pl.BlockSpec((BLK, FEAT), lambda i: (i, 0)),
            full((FEAT, HH)), full((1, HH)), full((HH, H)), full((1, H)),
            full((H, H)), full((FEAT, H)), full((1, H)), full((1, H)),
            full((H, 1)),
        ],
        out_specs=[
            pl.BlockSpec((BLK, HH), lambda i: (i, 0)),
            pl.BlockSpec((BLK, HH), lambda i: (i, 0)),
            pl.BlockSpec((BLK, H), lambda i: (i, 0)),
            pl.BlockSpec((BLK, 1), lambda i: (i, 0)),
            pl.BlockSpec((BLK, 1), lambda i: (i, 0)),
            pl.BlockSpec((1, 128), lambda i: (0, 0)),
        ],
        out_shape=[
            jax.ShapeDtypeStruct((NP, HH), jnp.float32),
            jax.ShapeDtypeStruct((NP, HH), jnp.float32),
            jax.ShapeDtypeStruct((NP, H), jnp.float32),
            jax.ShapeDtypeStruct((NP, 1), jnp.float32),
            jax.ShapeDtypeStruct((NP, 1), jnp.float32),
            jax.ShapeDtypeStruct((1, 128), jnp.float32),
        ],
    )(featp, W1, b1, W2, b2, Wh, Wf, bnr, cbit, a_m)


def _passa_body(src_hbm, dst_hbm, bit_hbm, aa_hbm, abp_hbm, consts_hbm,
                out_hbm, q_hbm,
                aa_v, abp_v, consts_v, srcb, dstb, bitb, qb, ridv, qtab, qsh):
    c = lax.axis_index("c")
    s = lax.axis_index("s")
    w = s * NC + c
    epert = E // (NC * NS)
    base_e = w * epert

    pltpu.sync_copy(aa_hbm, aa_v)
    pltpu.sync_copy(abp_hbm, abp_v)
    pltpu.sync_copy(consts_hbm, consts_v)

    @pl.loop(0, AROW)
    def _zq(i):
        for t in range(128 // 16):
            qtab[i, pl.ds(t * 16, 16)] = jnp.zeros((16,), jnp.float32)

    for g in range(AROW // 16):
        ridv[pl.ds(g * 16, 16)] = lax.iota(jnp.int32, 16) + g * 16

    @pl.when(s < 5)
    def _ztab():
        pltpu.sync_copy(qtab.at[pl.ds(0, 16)], qsh.at[pl.ds(s * 16, 16)])

    plsc.subcore_barrier()

    cv = consts_v[pl.ds(0, 16)]
    g = cv[0] + jnp.maximum(cv[1], 0.0)
    c1 = cv[1]

    @pl.loop(0, epert // CH)
    def _chunk(k):
        base = base_e + k * CH
        pltpu.sync_copy(src_hbm.at[pl.ds(base, CH)], srcb)
        pltpu.sync_copy(dst_hbm.at[pl.ds(base, CH)], dstb)
        pltpu.sync_copy(bit_hbm.at[pl.ds(base, CH)], bitb)
        for grp in range(CH // 16):
            idxs = srcb[pl.ds(grp * 16, 16)]
            idxd = dstb[pl.ds(grp * 16, 16)]
            bitv = bitb[pl.ds(grp * 16, 16)]
            av = plsc.load_gather(aa_v, [idxs])
            bv = plsc.load_gather(abp_v, [idxd])
            u = av + bv + c1 * bitv
            e = jnp.maximum(u, 0.2 * u)
            capv = _lrelu(g + bv, 0.2)
            q = jnp.exp((e - capv) * 0.25)
            qb[pl.ds(grp * 16, 16)] = q
            plsc.addupdate_scatter(
                qtab, [lax.shift_right_logical(idxd, 7),
                       jnp.bitwise_and(idxd, 127)], q)
        pltpu.sync_copy(qb, q_hbm.at[pl.ds(base, CH)])

    pltpu.sync_copy(qtab, qsh.at[ridv], add=True)
    plsc.subcore_barrier()

    @pl.when(s < 5)
    def _out():
        pltpu.sync_copy(qsh.at[pl.ds(s * 16, 16)],
                        out_hbm.at[c, pl.ds(s * 16, 16)])


def _run_passa(src, dst, bit, aa, abp, consts):
    mesh = plsc.VectorSubcoreMesh(core_axis_name="c", subcore_axis_name="s",
                                  num_cores=NC, num_subcores=NS)
    f = pl.kernel(
        _passa_body,
        out_type=(jax.ShapeDtypeStruct((NC, AROW, 128), jnp.float32),
                  jax.ShapeDtypeStruct((E,), jnp.float32)),
        mesh=mesh,
        compiler_params=pltpu.CompilerParams(needs_layout_passes=False),
        scratch_types=[
            pltpu.VMEM((NP,), jnp.float32),
            pltpu.VMEM((NP,), jnp.float32),
            pltpu.VMEM((128,), jnp.float32),
            pltpu.VMEM((CH,), jnp.int32),
            pltpu.VMEM((CH,), jnp.int32),
            pltpu.VMEM((CH,), jnp.float32),
            pltpu.VMEM((CH,), jnp.float32),
            pltpu.VMEM((AROW,), jnp.int32),
            pltpu.VMEM((AROW, 128), jnp.float32),
            pltpu.VMEM_SHARED((AROW, 128), jnp.float32),
        ],
    )
    return f(src, dst, bit, aa, abp, consts)


def _mid_body(s0a0_ref, s0a1_ref, sinv_ref):
    stot = s0a0_ref[0] + s0a1_ref[0]
    pos = stot > 0
    sinv_ref[...] = jnp.where(pos, 1.0 / jnp.where(pos, stot, 1.0), 0.0)


def _run_mid(s0a):
    return pl.pallas_call(
        _mid_body,
        grid=(1,),
        in_specs=[
            pl.BlockSpec((1, AROW, 128), lambda i: (0, 0, 0)),
            pl.BlockSpec((1, AROW, 128), lambda i: (1, 0, 0)),
        ],
        out_specs=pl.BlockSpec((AROW, 128), lambda i: (0, 0)),
        out_shape=jax.ShapeDtypeStruct((AROW, 128), jnp.float32),
    )(s0a, s0a)


def _passb_body(src_hbm, dst_hbm, bit_hbm, q_hbm, sinv_hbm, a0_hbm, a1_hbm,
                out_hbm, outw_hbm,
                sinv_v, srcb, dstb, bitb, qb, ridv, rid2v,
                rows, stab, table, wsh, sem):
    c = lax.axis_index("c")
    s = lax.axis_index("s")
    epert = E // NS
    base_e = s * epert

    pltpu.sync_copy(sinv_hbm, sinv_v)

    @pl.loop(0, CH)
    def _zero(i):
        for t in range(HH // 16):
            rows[i, pl.ds(t * 16, 16)] = jnp.zeros((16,), jnp.float32)

    @pl.loop(0, BROW)
    def _zs(i):
        for t in range(128 // 16):
            stab[i, pl.ds(t * 16, 16)] = jnp.zeros((16,), jnp.float32)

    for g in range(AROW // 16):
        rid = lax.iota(jnp.int32, 16) + g * 16
        ridv[pl.ds(g * 16, 16)] = rid
        rid2v[pl.ds(g * 16, 16)] = rid + AROW

    for j in range(ROWB // CH):
        pltpu.sync_copy(rows, table.at[pl.ds(s * ROWB + j * CH, CH)])

    @pl.when(s < BROW // 16)
    def _ztab():
        pltpu.sync_copy(stab.at[pl.ds(0, 16)], wsh.at[pl.ds(s * 16, 16)])

    plsc.subcore_barrier()

    @pl.loop(0, epert // CH)
    def _chunk(k):
        base = base_e + k * CH
        pltpu.sync_copy(src_hbm.at[pl.ds(base, CH)], srcb)
        pltpu.sync_copy(dst_hbm.at[pl.ds(base, CH)], dstb)
        pltpu.sync_copy(bit_hbm.at[pl.ds(base, CH)], bitb)
        pltpu.sync_copy(q_hbm.at[pl.ds(base, CH)], qb)

        @pl.when(c == 0)
        def _g0():
            pltpu.async_copy(a0_hbm.at[srcb], rows, sem).wait()

        @pl.when(c != 0)
        def _g1():
            pltpu.async_copy(a1_hbm.at[srcb], rows, sem).wait()

        for grp in range(CH // 16):
            idxd = dstb[pl.ds(grp * 16, 16)]
            bitv = bitb[pl.ds(grp * 16, 16)]
            q = qb[pl.ds(grp * 16, 16)]
            sv = plsc.load_gather(sinv_v, [idxd])
            t = q * sv
            t2 = t * t
            wv = t2 * t2
            row6 = lax.shift_right_logical(idxd, 6)
            wcol = jnp.bitwise_and(idxd, 63) * 2
            plsc.addupdate_scatter(stab, [row6, wcol], wv)
            plsc.addupdate_scatter(stab, [row6, wcol + 1], wv * bitv)
            for lane in range(16):
                ei = grp * 16 + lane
                wsc = wv[lane]
                for t in range(HH // 16):
                    rows[ei, pl.ds(t * 16, 16)] = (
                        rows[ei, pl.ds(t * 16, 16)] * wsc)
        pltpu.sync_copy(rows, table.at[dstb], add=True)

    @pl.when(c == 0)
    def _merge():
        pltpu.sync_copy(stab.at[pl.ds(0, AROW)], wsh.at[ridv], add=True)
        pltpu.sync_copy(stab.at[pl.ds(AROW, AROW)], wsh.at[rid2v], add=True)

    plsc.subcore_barrier()
    pltpu.sync_copy(table.at[pl.ds(s * ROWB, ROWB)],
                    out_hbm.at[c, pl.ds(s * ROWB, ROWB)])

    @pl.when(jnp.logical_and(c == 0, s < BROW // 16))
    def _outw():
        pltpu.sync_copy(wsh.at[pl.ds(s * 16, 16)],
                        outw_hbm.at[pl.ds(s * 16, 16)])


def _run_passb(src, dst, bit, q, sinv, a0, a1):
    mesh = plsc.VectorSubcoreMesh(core_axis_name="c", subcore_axis_name="s",
                                  num_cores=NC, num_subcores=NS)
    f = pl.kernel(
        _passb_body,
        out_type=(jax.ShapeDtypeStruct((NC, NP, HH), jnp.float32),
                  jax.ShapeDtypeStruct((BROW, 128), jnp.float32)),
        mesh=mesh,
        compiler_params=pltpu.CompilerParams(needs_layout_passes=False),
        scratch_types=[
            pltpu.VMEM((NP,), jnp.float32),
            pltpu.VMEM((CH,), jnp.int32),
            pltpu.VMEM((CH,), jnp.int32),
            pltpu.VMEM((CH,), jnp.float32),
            pltpu.VMEM((CH,), jnp.float32),
            pltpu.VMEM((AROW,), jnp.int32),
            pltpu.VMEM((AROW,), jnp.int32),
            pltpu.VMEM((CH, HH), jnp.float32),
            pltpu.VMEM((BROW, 128), jnp.float32),
            pltpu.VMEM_SHARED((NP, HH), jnp.float32),
            pltpu.VMEM_SHARED((BROW, 128), jnp.float32),
            pltpu.SemaphoreType.DMA,
        ],
    )
    return f(src, dst, bit, q, sinv, a0, a1)


def _post_body(sa0_ref, sa1_ref, s0_ref, s1_ref, bp_ref, cbit_ref, wo1_ref,
               bo1_ref, wo2_ref, bo2_ref, out_ref):
    s0 = s0_ref[...]
    s1 = s1_ref[...]
    sa = jnp.concatenate([sa0_ref[0], sa1_ref[0]], axis=1)
    inv = 1.0 / jnp.where(s0 > 0, s0, 1.0)
    neigh = jnp.where(s0 > 0, (sa + s1 * cbit_ref[...]) * inv + bp_ref[...], 0.0)
    ho = jnp.maximum(neigh, 0.0)
    t = jnp.dot(ho, wo1_ref[...], preferred_element_type=jnp.float32) + bo1_ref[...]
    t = _lrelu(t, 0.1)
    out_ref[...] = jnp.dot(t, wo2_ref[...], preferred_element_type=jnp.float32) + bo2_ref[...]


def _run_post(sab, s0, s1, bp, cbit, Wo1, bo1, Wo2, bo2):
    nb = NP // BLK
    full = lambda s: pl.BlockSpec(s, lambda i: (0, 0))
    return pl.pallas_call(
        _post_body,
        grid=(nb,),
        in_specs=[
            pl.BlockSpec((1, BLK, HH), lambda i: (0, i, 0)),
            pl.BlockSpec((1, BLK, HH), lambda i: (1, i, 0)),
            pl.BlockSpec((BLK, 1), lambda i: (i, 0)),
            pl.BlockSpec((BLK, 1), lambda i: (i, 0)),
            pl.BlockSpec((BLK, H), lambda i: (i, 0)),
            full((1, H)), full((H, H)), full((1, H)), full((H, 1)),
            full((1, 1)),
        ],
        out_specs=pl.BlockSpec((BLK, 1), lambda i: (i, 0)),
        out_shape=jax.ShapeDtypeStruct((NP, 1), jnp.float32),
    )(sab, sab, s0, s1, bp, cbit, Wo1, bo1, Wo2, bo2)


@jax.jit
def kernel(feat, bit_position, edge_index, W_self1, b_self1, W_self2, b_self2,
           Wn, bn, a_m, Wo1, bo1, Wo2, bo2):
    src = edge_index[0]
    dst = edge_index[1]
    bit = bit_position[:, 0]
    featp = jnp.pad(feat, ((0, NP - N), (0, 0)))
    Wh = Wn[:H]
    cbit = Wn[H:H + 1]
    Wf = Wn[H + 1:]
    a0, a1, bp, aa2, abp2, consts = _run_pre(
        featp, W_self1, b_self1[None, :], W_self2, b_self2[None, :],
        Wh, Wf, bn[None, :], cbit, a_m)
    aa = aa2[:, 0]
    abp = abp2[:, 0]
    constsv = consts[0]
    s0a, q = _run_passa(src, dst, bit, aa, abp, constsv)
    sinv = _run_mid(s0a).reshape(NP)
    sab, sw = _run_passb(src, dst, bit, q, sinv, a0, a1)
    sw2 = sw.reshape(NP, 2)
    out = _run_post(sab, sw2[:, 0:1], sw2[:, 1:2], bp, cbit,
                    Wo1, bo1[None, :], Wo2, bo2[None, :])
    return out[:N]


# speedup vs baseline: 9.7445x; 1.3469x over previous
import jax
import jax.numpy as jnp
from jax import lax
from jax.experimental import pallas as pl
from jax.experimental.pallas import tpu as pltpu
from jax.experimental.pallas import tpu_sc as plsc

N = 10000
NP = 10240
E = 320000
FEAT = 128
H = 256
HH = 128
NC = 2
NS = 16
CH = 80
CHA = 400
ROWB = NP // NS
BLK = 1280
AROW = NP // 128
BROW = NP // 64


def _lrelu(x, s):
    return jnp.maximum(x, s * x)


def _pre_body(x_ref, w1_ref, b1_ref, w2_ref, b2_ref, wh_ref, wf_ref, bn_ref,
              cbit_ref, am_ref,
              a0_ref, a1_ref, bp_ref, aa_ref, abp_ref, consts_ref):
    i = pl.program_id(0)
    x = x_ref[...]
    t = jnp.dot(x, w1_ref[...], preferred_element_type=jnp.float32) + b1_ref[...]
    t = _lrelu(t, 0.1)
    h = jnp.dot(t, w2_ref[...], preferred_element_type=jnp.float32) + b2_ref[...]
    a = jnp.dot(h, wh_ref[...], preferred_element_type=jnp.float32)
    bp = jnp.dot(x, wf_ref[...], preferred_element_type=jnp.float32) + bn_ref[...]
    a0_ref[...] = a[:, :HH]
    a1_ref[...] = a[:, HH:]
    bp_ref[...] = bp
    am = am_ref[...]
    aa = jnp.dot(a, am, preferred_element_type=jnp.float32)
    abp = jnp.dot(bp, am, preferred_element_type=jnp.float32)
    aa_ref[...] = aa
    abp_ref[...] = abp
    c1 = jnp.sum(cbit_ref[...] * am[:, 0][None, :])
    prevm = jnp.where(i == 0, -jnp.inf, consts_ref[0, 0])
    m = jnp.maximum(prevm, jnp.max(aa))
    lane = lax.broadcasted_iota(jnp.int32, (1, 128), 1)
    consts_ref[...] = jnp.where(lane == 0, m, jnp.where(lane == 1, c1, 0.0))


def _run_pre(featp, W1, b1, W2, b2, Wh, Wf, bnr, cbit, a_m):
    nb = NP // BLK
    full = lambda s: pl.BlockSpec(s, lambda i: (0, 0))
    return pl.pallas_call(
        _pre_body,
        grid=(nb,),
        in_specs=[
            pl.BlockSpec((BLK, FEAT), lambda i: (i, 0)),
            full((FEAT, HH)), full((1, HH)), full((HH, H)), full((1, H)),
            full((H, H)), full((FEAT, H)), full((1, H)), full((1, H)),
            full((H, 1)),
        ],
        out_specs=[
            pl.BlockSpec((BLK, HH), lambda i: (i, 0)),
            pl.BlockSpec((BLK, HH), lambda i: (i, 0)),
            pl.BlockSpec((BLK, H), lambda i: (i, 0)),
            pl.BlockSpec((BLK, 1), lambda i: (i, 0)),
            pl.BlockSpec((BLK, 1), lambda i: (i, 0)),
            pl.BlockSpec((1, 128), lambda i: (0, 0)),
        ],
        out_shape=[
            jax.ShapeDtypeStruct((NP, HH), jnp.float32),
            jax.ShapeDtypeStruct((NP, HH), jnp.float32),
            jax.ShapeDtypeStruct((NP, H), jnp.float32),
            jax.ShapeDtypeStruct((NP, 1), jnp.float32),
            jax.ShapeDtypeStruct((NP, 1), jnp.float32),
            jax.ShapeDtypeStruct((1, 128), jnp.float32),
        ],
    )(featp, W1, b1, W2, b2, Wh, Wf, bnr, cbit, a_m)


def _passa_body(src_hbm, dst_hbm, bit_hbm, aa_hbm, abp_hbm, consts_hbm,
                out_hbm, q_hbm,
                aa_v, abp_v, consts_v, srcb, dstb, bitb, qb, ridv, qtab, qsh):
    c = lax.axis_index("c")
    s = lax.axis_index("s")
    w = s * NC + c
    epert = E // (NC * NS)
    base_e = w * epert

    pltpu.sync_copy(aa_hbm, aa_v)
    pltpu.sync_copy(abp_hbm, abp_v)
    pltpu.sync_copy(consts_hbm, consts_v)

    @pl.loop(0, AROW)
    def _zq(i):
        for t in range(128 // 16):
            qtab[i, pl.ds(t * 16, 16)] = jnp.zeros((16,), jnp.float32)

    for g in range(AROW // 16):
        ridv[pl.ds(g * 16, 16)] = lax.iota(jnp.int32, 16) + g * 16

    @pl.when(s < 5)
    def _ztab():
        pltpu.sync_copy(qtab.at[pl.ds(0, 16)], qsh.at[pl.ds(s * 16, 16)])

    plsc.subcore_barrier()

    cv = consts_v[pl.ds(0, 16)]
    g = cv[0] + jnp.maximum(cv[1], 0.0)
    c1 = cv[1]

    @pl.loop(0, epert // CHA)
    def _chunk(k):
        base = base_e + k * CHA
        pltpu.sync_copy(src_hbm.at[pl.ds(base, CHA)], srcb)
        pltpu.sync_copy(dst_hbm.at[pl.ds(base, CHA)], dstb)
        pltpu.sync_copy(bit_hbm.at[pl.ds(base, CHA)], bitb)
        for grp in range(CHA // 16):
            idxs = srcb[pl.ds(grp * 16, 16)]
            idxd = dstb[pl.ds(grp * 16, 16)]
            bitv = bitb[pl.ds(grp * 16, 16)]
            av = plsc.load_gather(aa_v, [idxs])
            bv = plsc.load_gather(abp_v, [idxd])
            u = av + bv + c1 * bitv
            e = jnp.maximum(u, 0.2 * u)
            capv = _lrelu(g + bv, 0.2)
            q = jnp.exp((e - capv) * 0.25)
            qb[pl.ds(grp * 16, 16)] = q
            plsc.addupdate_scatter(
                qtab, [lax.shift_right_logical(idxd, 7),
                       jnp.bitwise_and(idxd, 127)], q)
        pltpu.sync_copy(qb, q_hbm.at[pl.ds(base, CHA)])

    pltpu.sync_copy(qtab, qsh.at[ridv], add=True)
    plsc.subcore_barrier()

    @pl.when(s < 5)
    def _out():
        pltpu.sync_copy(qsh.at[pl.ds(s * 16, 16)],
                        out_hbm.at[c, pl.ds(s * 16, 16)])


def _run_passa(src, dst, bit, aa, abp, consts):
    mesh = plsc.VectorSubcoreMesh(core_axis_name="c", subcore_axis_name="s",
                                  num_cores=NC, num_subcores=NS)
    f = pl.kernel(
        _passa_body,
        out_type=(jax.ShapeDtypeStruct((NC, AROW, 128), jnp.float32),
                  jax.ShapeDtypeStruct((E,), jnp.float32)),
        mesh=mesh,
        compiler_params=pltpu.CompilerParams(needs_layout_passes=False),
        scratch_types=[
            pltpu.VMEM((NP,), jnp.float32),
            pltpu.VMEM((NP,), jnp.float32),
            pltpu.VMEM((128,), jnp.float32),
            pltpu.VMEM((CHA,), jnp.int32),
            pltpu.VMEM((CHA,), jnp.int32),
            pltpu.VMEM((CHA,), jnp.float32),
            pltpu.VMEM((CHA,), jnp.float32),
            pltpu.VMEM((AROW,), jnp.int32),
            pltpu.VMEM((AROW, 128), jnp.float32),
            pltpu.VMEM_SHARED((AROW, 128), jnp.float32),
        ],
    )
    return f(src, dst, bit, aa, abp, consts)


def _mid_body(s0a0_ref, s0a1_ref, sinv_ref):
    stot = s0a0_ref[0] + s0a1_ref[0]
    pos = stot > 0
    sinv_ref[...] = jnp.where(pos, 1.0 / jnp.where(pos, stot, 1.0), 0.0)


def _run_mid(s0a):
    return pl.pallas_call(
        _mid_body,
        grid=(1,),
        in_specs=[
            pl.BlockSpec((1, AROW, 128), lambda i: (0, 0, 0)),
            pl.BlockSpec((1, AROW, 128), lambda i: (1, 0, 0)),
        ],
        out_specs=pl.BlockSpec((AROW, 128), lambda i: (0, 0)),
        out_shape=jax.ShapeDtypeStruct((AROW, 128), jnp.float32),
    )(s0a, s0a)


def _passb_body(src_hbm, dst_hbm, bit_hbm, q_hbm, sinv_hbm, a0_hbm, a1_hbm,
                out_hbm, outw_hbm,
                sinv_v, srcb0, dstb0, bitb0, qb0, srcb1, dstb1, bitb1, qb1,
                ridv, rows0, rows1, stab, table, wsh,
                gsem0, gsem1, ssem0, ssem1):
    c = lax.axis_index("c")
    s = lax.axis_index("s")
    epert = E // NS
    base_e = s * epert
    half = (epert // CH) // 2

    pltpu.sync_copy(sinv_hbm, sinv_v)

    @pl.loop(0, CH)
    def _zero(i):
        for t in range(HH // 16):
            rows0[i, pl.ds(t * 16, 16)] = jnp.zeros((16,), jnp.float32)

    @pl.loop(0, AROW)
    def _zs(i):
        for t in range(128 // 16):
            stab[i, pl.ds(t * 16, 16)] = jnp.zeros((16,), jnp.float32)

    for g in range(AROW // 16):
        ridv[pl.ds(g * 16, 16)] = lax.iota(jnp.int32, 16) + g * 16

    for j in range(ROWB // CH):
        pltpu.sync_copy(rows0, table.at[pl.ds(s * ROWB + j * CH, CH)])

    @pl.when(s < AROW // 16)
    def _ztab():
        pltpu.sync_copy(stab.at[pl.ds(0, 16)], wsh.at[pl.ds(s * 16, 16)])

    plsc.subcore_barrier()

    def load_slabs(base, sb, db, bb, qbuf):
        pltpu.sync_copy(src_hbm.at[pl.ds(base, CH)], sb)
        pltpu.sync_copy(dst_hbm.at[pl.ds(base, CH)], db)
        pltpu.sync_copy(bit_hbm.at[pl.ds(base, CH)], bb)
        pltpu.sync_copy(q_hbm.at[pl.ds(base, CH)], qbuf)

    def issue_gather(sb, rbuf, gsem):
        @pl.when(c == 0)
        def _g0():
            pltpu.async_copy(a0_hbm.at[sb], rbuf, gsem)

        @pl.when(c != 0)
        def _g1():
            pltpu.async_copy(a1_hbm.at[sb], rbuf, gsem)

    def wait_gather(sb, rbuf, gsem):
        @pl.when(c == 0)
        def _w0():
            pltpu.make_async_copy(a0_hbm.at[sb], rbuf, gsem).wait()

        @pl.when(c != 0)
        def _w1():
            pltpu.make_async_copy(a1_hbm.at[sb], rbuf, gsem).wait()

    def issue_scatter(rbuf, db, ssem):
        pltpu.async_copy(rbuf, table.at[db], ssem, add=True)

    def wait_scatter(rbuf, db, ssem):
        pltpu.make_async_copy(rbuf, table.at[db], ssem).wait()

    def compute(db, bb, qbuf, rbuf):
        for grp in range(CH // 16):
            idxd = db[pl.ds(grp * 16, 16)]
            bitv = bb[pl.ds(grp * 16, 16)]
            q = qbuf[pl.ds(grp * 16, 16)]
            sv = plsc.load_gather(sinv_v, [idxd])
            t = q * sv
            t2 = t * t
            wv = t2 * t2
            val = jnp.where(c == 0, wv, wv * bitv)
            plsc.addupdate_scatter(
                stab, [lax.shift_right_logical(idxd, 7),
                       jnp.bitwise_and(idxd, 127)], val)
            for lane in range(16):
                ei = grp * 16 + lane
                wsc = wv[lane]
                for t in range(HH // 16):
                    rbuf[ei, pl.ds(t * 16, 16)] = (
                        rbuf[ei, pl.ds(t * 16, 16)] * wsc)

    load_slabs(base_e, srcb0, dstb0, bitb0, qb0)
    issue_gather(srcb0, rows0, gsem0)

    @pl.loop(0, half)
    def _pair(kk):
        ebase = base_e + kk * (2 * CH)
        wait_gather(srcb0, rows0, gsem0)

        @pl.when(kk > 0)
        def _ws1():
            wait_scatter(rows1, dstb1, ssem1)

        load_slabs(ebase + CH, srcb1, dstb1, bitb1, qb1)
        issue_gather(srcb1, rows1, gsem1)
        compute(dstb0, bitb0, qb0, rows0)
        issue_scatter(rows0, dstb0, ssem0)
        wait_gather(srcb1, rows1, gsem1)

        @pl.when(kk < half - 1)
        def _pre0():
            wait_scatter(rows0, dstb0, ssem0)
            load_slabs(ebase + 2 * CH, srcb0, dstb0, bitb0, qb0)
            issue_gather(srcb0, rows0, gsem0)

        compute(dstb1, bitb1, qb1, rows1)
        issue_scatter(rows1, dstb1, ssem1)

    wait_scatter(rows0, dstb0, ssem0)
    wait_scatter(rows1, dstb1, ssem1)

    pltpu.sync_copy(stab, wsh.at[ridv], add=True)

    plsc.subcore_barrier()
    pltpu.sync_copy(table.at[pl.ds(s * ROWB, ROWB)],
                    out_hbm.at[c, pl.ds(s * ROWB, ROWB)])

    @pl.when(s < AROW // 16)
    def _outw():
        pltpu.sync_copy(wsh.at[pl.ds(s * 16, 16)],
                        outw_hbm.at[c, pl.ds(s * 16, 16)])


def _run_passb(src, dst, bit, q, sinv, a0, a1):
    mesh = plsc.VectorSubcoreMesh(core_axis_name="c", subcore_axis_name="s",
                                  num_cores=NC, num_subcores=NS)
    f = pl.kernel(
        _passb_body,
        out_type=(jax.ShapeDtypeStruct((NC, NP, HH), jnp.float32),
                  jax.ShapeDtypeStruct((NC, AROW, 128), jnp.float32)),
        mesh=mesh,
        compiler_params=pltpu.CompilerParams(needs_layout_passes=False),
        scratch_types=[
            pltpu.VMEM((NP,), jnp.float32),
            pltpu.VMEM((CH,), jnp.int32),
            pltpu.VMEM((CH,), jnp.int32),
            pltpu.VMEM((CH,), jnp.float32),
            pltpu.VMEM((CH,), jnp.float32),
            pltpu.VMEM((CH,), jnp.int32),
            pltpu.VMEM((CH,), jnp.int32),
            pltpu.VMEM((CH,), jnp.float32),
            pltpu.VMEM((CH,), jnp.float32),
            pltpu.VMEM((AROW,), jnp.int32),
            pltpu.VMEM((CH, HH), jnp.float32),
            pltpu.VMEM((CH, HH), jnp.float32),
            pltpu.VMEM((AROW, 128), jnp.float32),
            pltpu.VMEM_SHARED((NP, HH), jnp.float32),
            pltpu.VMEM_SHARED((AROW, 128), jnp.float32),
            pltpu.SemaphoreType.DMA,
            pltpu.SemaphoreType.DMA,
            pltpu.SemaphoreType.DMA,
            pltpu.SemaphoreType.DMA,
        ],
    )
    return f(src, dst, bit, q, sinv, a0, a1)


def _post_body(sa0_ref, sa1_ref, s0_ref, s1_ref, bp_ref, cbit_ref, wo1_ref,
               bo1_ref, wo2_ref, bo2_ref, out_ref):
    s0 = s0_ref[...]
    s1 = s1_ref[...]
    sa = jnp.concatenate([sa0_ref[0], sa1_ref[0]], axis=1)
    inv = 1.0 / jnp.where(s0 > 0, s0, 1.0)
    neigh = jnp.where(s0 > 0, (sa + s1 * cbit_ref[...]) * inv + bp_ref[...], 0.0)
    ho = jnp.maximum(neigh, 0.0)
    t = jnp.dot(ho, wo1_ref[...], preferred_element_type=jnp.float32) + bo1_ref[...]
    t = _lrelu(t, 0.1)
    out_ref[...] = jnp.dot(t, wo2_ref[...], preferred_element_type=jnp.float32) + bo2_ref[...]


def _run_post(sab, s0, s1, bp, cbit, Wo1, bo1, Wo2, bo2):
    nb = NP // BLK
    full = lambda s: pl.BlockSpec(s, lambda i: (0, 0))
    return pl.pallas_call(
        _post_body,
        grid=(nb,),
        in_specs=[
            pl.BlockSpec((1, BLK, HH), lambda i: (0, i, 0)),
            pl.BlockSpec((1, BLK, HH), lambda i: (1, i, 0)),
            pl.BlockSpec((BLK, 1), lambda i: (i, 0)),
            pl.BlockSpec((BLK, 1), lambda i: (i, 0)),
            pl.BlockSpec((BLK, H), lambda i: (i, 0)),
            full((1, H)), full((H, H)), full((1, H)), full((H, 1)),
            full((1, 1)),
        ],
        out_specs=pl.BlockSpec((BLK, 1), lambda i: (i, 0)),
        out_shape=jax.ShapeDtypeStruct((NP, 1), jnp.float32),
    )(sab, sab, s0, s1, bp, cbit, Wo1, bo1, Wo2, bo2)


@jax.jit
def kernel(feat, bit_position, edge_index, W_self1, b_self1, W_self2, b_self2,
           Wn, bn, a_m, Wo1, bo1, Wo2, bo2):
    src = edge_index[0]
    dst = edge_index[1]
    bit = bit_position[:, 0]
    featp = jnp.pad(feat, ((0, NP - N), (0, 0)))
    Wh = Wn[:H]
    cbit = Wn[H:H + 1]
    Wf = Wn[H + 1:]
    a0, a1, bp, aa2, abp2, consts = _run_pre(
        featp, W_self1, b_self1[None, :], W_self2, b_self2[None, :],
        Wh, Wf, bn[None, :], cbit, a_m)
    aa = aa2[:, 0]
    abp = abp2[:, 0]
    constsv = consts[0]
    s0a, q = _run_passa(src, dst, bit, aa, abp, constsv)
    sinv = _run_mid(s0a).reshape(NP)
    sab, sw = _run_passb(src, dst, bit, q, sinv, a0, a1)
    sw2 = sw.reshape(NP, 2)
    out = _run_post(sab, sw2[:, 0:1], sw2[:, 1:2], bp, cbit,
                    Wo1, bo1[None, :], Wo2, bo2[None, :])
    return out[:N]
